# Initial kernel scaffold; baseline (speedup 1.0000x reference)
import jax
import jax.numpy as jnp
from jax.experimental import pallas as pl


def kernel(samples, targets):
    raise NotImplementedError("write your pallas kernel here")


# speedup vs baseline: 298.7211x; 298.7211x over previous
import numpy as np
import jax
import jax.numpy as jnp
from jax import lax
from jax.experimental import pallas as pl
from jax.experimental.pallas import tpu as pltpu
from jax.experimental.pallas import tpu_sc as plsc

MARGIN = 0.15
EPS = 1e-6
N = 512
FEAT = 128
L = 16
NCHUNK = N // L
WIN_ROWS = 88
STREAM_WORDS = 2 ** 20 + 4096
W_ROWS = STREAM_WORDS // L

_W_HOST = (
    np.random.RandomState(0)
    .randint(0, 2 ** 32, size=STREAM_WORDS, dtype=np.uint32)
    .astype(np.int32)
    .reshape(W_ROWS, L)
)


def _tc_body(s_ref, trow_ref, tcol_ref, d_ref, npos_ref, nneg_ref, cnt_ref):
    s = s_ref[...]
    dn = (((1,), (1,)), ((), ()))
    g = lax.dot_general(s, s, dn, precision=lax.Precision.HIGHEST,
                        preferred_element_type=jnp.float32)
    ssq = s * s
    ones_row = jnp.ones((1, FEAT), jnp.float32)
    sq_row = jnp.sum(ssq, axis=1, keepdims=True)
    sq_col = lax.dot_general(ones_row, ssq, dn,
                             precision=lax.Precision.HIGHEST,
                             preferred_element_type=jnp.float32)
    rs_row = jnp.sum(s, axis=1, keepdims=True)
    rs_col = lax.dot_general(ones_row, s, dn,
                             precision=lax.Precision.HIGHEST,
                             preferred_element_type=jnp.float32)
    d2 = (sq_row + sq_col - 2.0 * g
          + (2.0 * EPS) * (rs_row - rs_col)
          + jnp.float32(FEAT * EPS * EPS))
    d_ref[...] = jnp.sqrt(jnp.maximum(d2, 0.0))

    trow = trow_ref[...]
    tcol = tcol_ref[...]
    rowi = lax.broadcasted_iota(jnp.int32, (N, N), 0)
    colj = lax.broadcasted_iota(jnp.int32, (N, N), 1)
    eq = trow == tcol
    posm = jnp.logical_and(eq, colj > rowi).astype(jnp.float32)
    negm = jnp.logical_not(eq).astype(jnp.float32)
    nposi = jnp.sum(posm, axis=1, keepdims=True).astype(jnp.int32)
    nnegi = jnp.sum(negm, axis=1, keepdims=True).astype(jnp.int32)
    npos_ref[...] = nposi
    nneg_ref[...] = nnegi
    valid = jnp.logical_and(nposi > 0, nnegi > 0)
    m = jnp.where(valid, jnp.maximum(nposi, nnegi), 0)
    cnt_ref[...] = jnp.sum(m, axis=(0, 1), keepdims=True)


def _tc_stage(samples, trow, tcol):
    return pl.pallas_call(
        _tc_body,
        out_shape=[
            jax.ShapeDtypeStruct((N, N), jnp.float32),
            jax.ShapeDtypeStruct((N, 1), jnp.int32),
            jax.ShapeDtypeStruct((N, 1), jnp.int32),
            jax.ShapeDtypeStruct((1, 1), jnp.int32),
        ],
    )(samples, trow, tcol)


def _mask_of(kmin):
    v = kmin - 1
    v = v | (v >> 1)
    v = v | (v >> 2)
    v = v | (v >> 4)
    v = v | (v >> 8)
    return v | (v >> 16)


def _sc_body(d_hbm, tgt_hbm, npos_hbm, nneg_hbm, w_hbm, out_hbm,
             tgt_v, npos_v, nneg_v, offs_v, drow_v, dpos_v, dneg_v,
             win_v, acc_v, offs_sh):
    c = lax.axis_index("c")
    s = lax.axis_index("s")
    wid = c * 16 + s
    lanes = lax.iota(jnp.int32, L)

    def sread(ref, i):
        base = (i // L) * L
        v = ref[pl.ds(base, L)]
        return jnp.sum(jnp.where(lanes == i - base, v, jnp.zeros_like(v)))

    pltpu.sync_copy(tgt_hbm, tgt_v)
    pltpu.sync_copy(npos_hbm, npos_v)
    pltpu.sync_copy(nneg_hbm, nneg_v)

    @pl.when(s == 0)
    def _scan():
        def anchor_step(i, o):
            np_i = sread(npos_v, i)
            nn_i = sread(nneg_v, i)
            valid = jnp.logical_and(np_i > 0, nn_i > 0)
            kmin = jnp.maximum(jnp.minimum(np_i, nn_i), 1)
            need = jnp.logical_and(valid,
                                   jnp.logical_and(np_i != nn_i, kmin > 1))
            plsc.store_scatter(offs_v, [jnp.full((L,), i, jnp.int32)],
                               jnp.full((L,), o, jnp.int32), mask=lanes == 0)

            def consume(o):
                d = jnp.maximum(np_i, nn_i)
                mb = _mask_of(kmin)
                kmv = kmin - 1
                row0 = pl.multiple_of((o // (L * 8)) * 8, 8)
                base = row0 * L
                pltpu.sync_copy(w_hbm.at[pl.ds(row0, WIN_ROWS)], win_v)

                def cond(st):
                    k, cnt, _ = st
                    return jnp.logical_and(cnt < d, k < WIN_ROWS)

                def step(st):
                    k, cnt, fp = st
                    w = win_v[k, :]
                    flat = base + k * L + lanes
                    a = jnp.logical_and((w & mb) <= kmv, flat >= o)
                    ai = a.astype(jnp.int32)
                    cs = plsc.cumsum(ai)
                    ca = jnp.sum(ai)
                    hit = jnp.logical_and(a, cs == (d - cnt))
                    lane = jnp.max(plsc.all_reduce_ffs(hit))
                    fp = jnp.where(cnt + ca >= d, base + k * L + lane + 1, fp)
                    return k + 1, cnt + ca, fp

                _, _, fp = lax.while_loop(cond, step,
                                          (jnp.int32(0), jnp.int32(0), o))
                return fp

            return lax.cond(need, consume, lambda o: o, o)

        lax.fori_loop(0, N, anchor_step, jnp.int32(0))
        pltpu.sync_copy(offs_v, offs_sh)

    plsc.subcore_barrier()
    pltpu.sync_copy(offs_sh, offs_v)

    def do_anchor(a, acc):
        i = wid * 16 + a
        np_i = sread(npos_v, i)
        nn_i = sread(nneg_v, i)
        lbl = sread(tgt_v, i)
        valid = jnp.logical_and(np_i > 0, nn_i > 0)

        def run(acc):
            m = jnp.maximum(np_i, nn_i)
            kmin = jnp.maximum(jnp.minimum(np_i, nn_i), 1)
            unequal = np_i != nn_i
            need = jnp.logical_and(unequal, kmin > 1)
            pos_rand = jnp.logical_and(unequal, np_i < nn_i)
            neg_rand = jnp.logical_and(unequal, nn_i < np_i)

            pltpu.sync_copy(d_hbm.at[i], drow_v)

            def comp(ch, carry):
                pc, nc = carry
                t = tgt_v[pl.ds(ch * L, L)]
                col = ch * L + lanes
                pm = jnp.logical_and(t == lbl, col > i)
                nm = t != lbl
                dv = drow_v[pl.ds(ch * L, L)]
                pmi = pm.astype(jnp.int32)
                nmi = nm.astype(jnp.int32)
                pr = plsc.cumsum(pmi)
                nr = plsc.cumsum(nmi)
                plsc.store_scatter(dpos_v, [pc + pr - 1], dv, mask=pm)
                plsc.store_scatter(dneg_v, [nc + nr - 1], dv, mask=nm)
                return pc + jnp.sum(pmi), nc + jnp.sum(nmi)

            lax.fori_loop(0, NCHUNK, comp, (jnp.int32(0), jnp.int32(0)))

            def stream_terms(acc):
                o = sread(offs_v, i)
                mb = _mask_of(kmin)
                kmv = kmin - 1
                row0 = pl.multiple_of((o // (L * 8)) * 8, 8)
                base = row0 * L
                pltpu.sync_copy(w_hbm.at[pl.ds(row0, WIN_ROWS)], win_v)

                def cond(st):
                    k, cnt, _ = st
                    return jnp.logical_and(cnt < m, k < WIN_ROWS)

                def step(st):
                    k, cnt, acc = st
                    w = win_v[k, :]
                    flat = base + k * L + lanes
                    a = jnp.logical_and((w & mb) <= kmv, flat >= o)
                    ai = a.astype(jnp.int32)
                    jl = cnt + plsc.cumsum(ai) - 1
                    act = jnp.logical_and(a, jl < m)
                    v = jnp.minimum(w & mb, N - 1)
                    jc = jnp.clip(jl, 0, N - 1)
                    pidx = jnp.where(pos_rand, v, jc)
                    qidx = jnp.where(pos_rand, jc, v)
                    pv = plsc.load_gather(dpos_v, [pidx], mask=act)
                    qv = plsc.load_gather(dneg_v, [qidx], mask=act)
                    t = jnp.where(act,
                                  jnp.maximum(pv - qv + MARGIN, 0.0), 0.0)
                    return k + 1, cnt + jnp.sum(ai), acc + t

                _, _, acc = lax.while_loop(cond, step,
                                           (jnp.int32(0), jnp.int32(0), acc))
                return acc

            def det_terms(acc):
                def step(ch, acc):
                    jl = ch * L + lanes
                    act = jl < m
                    jc = jnp.clip(jl, 0, N - 1)
                    zero = jnp.zeros((L,), jnp.int32)
                    pidx = jnp.where(pos_rand, zero, jc)
                    qidx = jnp.where(neg_rand, zero, jc)
                    pv = plsc.load_gather(dpos_v, [pidx], mask=act)
                    qv = plsc.load_gather(dneg_v, [qidx], mask=act)
                    t = jnp.where(act,
                                  jnp.maximum(pv - qv + MARGIN, 0.0), 0.0)
                    return acc + t

                nch = (m + L - 1) // L
                return lax.fori_loop(0, nch, step, acc)

            return lax.cond(need, stream_terms, det_terms, acc)

        return lax.cond(valid, run, lambda acc: acc, acc)

    acc = lax.fori_loop(0, 16, do_anchor, jnp.zeros((L,), jnp.float32))
    acc_v[...] = acc
    pltpu.sync_copy(acc_v, out_hbm.at[wid])


def _sc_stage(d_mat, targets, npos, nneg, w_stream):
    mesh = plsc.VectorSubcoreMesh(core_axis_name="c", subcore_axis_name="s")
    f = pl.kernel(
        _sc_body,
        out_type=jax.ShapeDtypeStruct((32, L), jnp.float32),
        mesh=mesh,
        compiler_params=pltpu.CompilerParams(needs_layout_passes=False),
        scratch_types=[
            pltpu.VMEM((N,), jnp.int32),
            pltpu.VMEM((N,), jnp.int32),
            pltpu.VMEM((N,), jnp.int32),
            pltpu.VMEM((N,), jnp.int32),
            pltpu.VMEM((N,), jnp.float32),
            pltpu.VMEM((N,), jnp.float32),
            pltpu.VMEM((N,), jnp.float32),
            pltpu.VMEM((WIN_ROWS, L), jnp.int32),
            pltpu.VMEM((L,), jnp.float32),
            pltpu.VMEM_SHARED((N,), jnp.int32),
        ],
    )
    return f(d_mat, targets, npos, nneg, w_stream)


def kernel(samples, targets):
    trow = targets.reshape(N, 1)
    tcol = targets.reshape(1, N)
    d_mat, nposc, nnegc, cnt = _tc_stage(samples, trow, tcol)
    w_stream = jnp.asarray(_W_HOST)
    partial = _sc_stage(d_mat, targets, nposc.reshape(N), nnegc.reshape(N),
                        w_stream)
    return jnp.sum(partial) / cnt[0, 0].astype(jnp.float32)


# speedup vs baseline: 895.1065x; 2.9965x over previous
import numpy as np
import jax
import jax.numpy as jnp
from jax import lax
from jax.experimental import pallas as pl
from jax.experimental.pallas import tpu as pltpu
from jax.experimental.pallas import tpu_sc as plsc

MARGIN = 0.15
EPS = 1e-6
N = 512
FEAT = 128
L = 16
NCHUNK = N // L
WIN128 = 24
BLK128 = 128
RING128 = 256
W128 = 8320
STREAM_WORDS = W128 * 128
APT = N // 32

_W_HOST = (
    np.random.RandomState(0)
    .randint(0, 2 ** 32, size=STREAM_WORDS, dtype=np.uint32)
    .astype(np.int32)
    .reshape(W128, 128)
)


def _tc_body(s_ref, trow_ref, tcol_ref, d_ref, npos_ref, nneg_ref, cnt_ref):
    s = s_ref[...]
    dn = (((1,), (1,)), ((), ()))
    g = lax.dot_general(s, s, dn, precision=lax.Precision.HIGHEST,
                        preferred_element_type=jnp.float32)
    ssq = s * s
    ones_row = jnp.ones((1, FEAT), jnp.float32)
    sq_row = jnp.sum(ssq, axis=1, keepdims=True)
    sq_col = lax.dot_general(ones_row, ssq, dn,
                             precision=lax.Precision.HIGHEST,
                             preferred_element_type=jnp.float32)
    rs_row = jnp.sum(s, axis=1, keepdims=True)
    rs_col = lax.dot_general(ones_row, s, dn,
                             precision=lax.Precision.HIGHEST,
                             preferred_element_type=jnp.float32)
    d2 = (sq_row + sq_col - 2.0 * g
          + (2.0 * EPS) * (rs_row - rs_col)
          + jnp.float32(FEAT * EPS * EPS))
    d_ref[...] = jnp.sqrt(jnp.maximum(d2, 0.0))

    trow = trow_ref[...]
    tcol = tcol_ref[...]
    rowi = lax.broadcasted_iota(jnp.int32, (N, N), 0)
    colj = lax.broadcasted_iota(jnp.int32, (N, N), 1)
    eq = trow == tcol
    posm = jnp.logical_and(eq, colj > rowi).astype(jnp.float32)
    negm = jnp.logical_not(eq).astype(jnp.float32)
    nposi = jnp.sum(posm, axis=1, keepdims=True).astype(jnp.int32)
    nnegi = jnp.sum(negm, axis=1, keepdims=True).astype(jnp.int32)
    npos_ref[...] = nposi
    nneg_ref[...] = nnegi
    valid = jnp.logical_and(nposi > 0, nnegi > 0)
    m = jnp.where(valid, jnp.maximum(nposi, nnegi), 0)
    cnt_ref[...] = jnp.sum(m, axis=(0, 1), keepdims=True)


def _tc_stage(samples, trow, tcol):
    return pl.pallas_call(
        _tc_body,
        out_shape=[
            jax.ShapeDtypeStruct((N, N), jnp.float32),
            jax.ShapeDtypeStruct((N, 1), jnp.int32),
            jax.ShapeDtypeStruct((N, 1), jnp.int32),
            jax.ShapeDtypeStruct((1, 1), jnp.int32),
        ],
    )(samples, trow, tcol)


def _mask_of(kmin):
    v = kmin - 1
    v = v | (v >> 1)
    v = v | (v >> 2)
    v = v | (v >> 4)
    v = v | (v >> 8)
    return v | (v >> 16)


def _sc_body(d_hbm, tgt_hbm, npos_hbm, nneg_hbm, w_hbm, out_hbm,
             tgt_v, npos_v, nneg_v, offs_v, drow_v, dpos_b, dneg_b,
             win_v, ring_v, acc_v, offs_sh):
    c = lax.axis_index("c")
    s = lax.axis_index("s")
    wid = c * 16 + s
    lanes = lax.iota(jnp.int32, L)

    def sread(ref, i):
        base = (i // L) * L
        v = ref[pl.ds(base, L)]
        return jnp.sum(jnp.where(lanes == i - base, v, jnp.zeros_like(v)))

    pltpu.sync_copy(tgt_hbm, tgt_v)
    pltpu.sync_copy(npos_hbm, npos_v)
    pltpu.sync_copy(nneg_hbm, nneg_v)

    def compact_anchor(a):
        i = wid * APT + a
        np_i = sread(npos_v, i)
        nn_i = sread(nneg_v, i)
        lbl = sread(tgt_v, i)
        valid = jnp.logical_and(np_i > 0, nn_i > 0)

        @pl.when(valid)
        def _():
            pltpu.sync_copy(d_hbm.at[i], drow_v)
            av = jnp.full((L,), a, jnp.int32)

            def comp(ch, carry):
                pc, nc = carry
                t = tgt_v[pl.ds(ch * L, L)]
                col = ch * L + lanes
                pm = jnp.logical_and(t == lbl, col > i)
                nm = t != lbl
                dv = drow_v[pl.ds(ch * L, L)]
                pmi = pm.astype(jnp.int32)
                nmi = nm.astype(jnp.int32)
                pr = plsc.cumsum(pmi)
                nr = plsc.cumsum(nmi)
                plsc.store_scatter(dpos_b, [av, pc + pr - 1], dv, mask=pm)
                plsc.store_scatter(dneg_b, [av, nc + nr - 1], dv, mask=nm)
                return pc + jnp.sum(pmi), nc + jnp.sum(nmi)

            lax.fori_loop(0, NCHUNK, comp, (jnp.int32(0), jnp.int32(0)))

    def ring_chunk(q):
        return ring_v[(q // 8) & (RING128 - 1), pl.ds((q & 7) * L, L)]

    @pl.when(s == 0)
    def _scan():
        for b in range(2):
            pltpu.sync_copy(w_hbm.at[pl.ds(b * BLK128, BLK128)],
                            ring_v.at[pl.ds(b * BLK128, BLK128)])

        def anchor_step(i, carry):
            o, loaded = carry
            np_i = sread(npos_v, i)
            nn_i = sread(nneg_v, i)
            valid = jnp.logical_and(np_i > 0, nn_i > 0)
            kmin = jnp.maximum(jnp.minimum(np_i, nn_i), 1)
            need = jnp.logical_and(valid,
                                   jnp.logical_and(np_i != nn_i, kmin > 1))
            plsc.store_scatter(offs_v, [jnp.full((L,), i, jnp.int32)],
                               jnp.full((L,), o, jnp.int32), mask=lanes == 0)

            def consume(carry):
                o, loaded = carry
                d = jnp.maximum(np_i, nn_i)
                mb = _mask_of(kmin)
                kmv = kmin - 1
                q0 = o // L
                phi = o - q0 * L
                r128 = o // 128

                def refill_cond(ld):
                    return r128 + WIN128 > ld

                def refill(ld):
                    slot = (ld // BLK128) % (RING128 // BLK128)
                    pltpu.sync_copy(
                        w_hbm.at[pl.ds(pl.multiple_of(ld, BLK128), BLK128)],
                        ring_v.at[pl.ds(pl.multiple_of(slot * BLK128,
                                                       BLK128), BLK128)])
                    return ld + BLK128

                loaded = lax.while_loop(refill_cond, refill, loaded)

                w = ring_chunk(q0)
                a0 = jnp.logical_and((w & mb) <= kmv, lanes < phi)
                dd = d + jnp.sum(a0.astype(jnp.int32))

                def cond(st):
                    k, _, cnt = st
                    return jnp.logical_and(cnt < dd, k < q0 + 80)

                def step(st):
                    k, _, cnt = st
                    ca = jnp.int32(0)
                    for u in range(4):
                        w = ring_chunk(k + u)
                        au = (w & mb) <= kmv
                        ca = ca + jnp.sum(au.astype(jnp.int32))
                    return k + 4, cnt, cnt + ca

                k, cprev, _ = lax.while_loop(
                    cond, step, (q0, jnp.int32(0), jnp.int32(0)))

                fp = o
                bc = cprev
                for u in range(4):
                    w = ring_chunk(k - 4 + u)
                    au = (w & mb) <= kmv
                    ai = au.astype(jnp.int32)
                    ca = jnp.sum(ai)
                    cs = plsc.cumsum(ai)
                    hit = jnp.logical_and(au, cs == dd - bc)
                    lane = jnp.max(plsc.all_reduce_ffs(hit))
                    crossed = jnp.logical_and(bc < dd, bc + ca >= dd)
                    fp = jnp.where(crossed, (k - 4 + u) * L + lane + 1, fp)
                    bc = bc + ca
                return fp, loaded

            return lax.cond(need, consume, lambda cr: cr, (o, loaded))

        lax.fori_loop(0, N, anchor_step,
                      (jnp.int32(0), jnp.int32(2 * BLK128)))
        pltpu.sync_copy(offs_v, offs_sh)

    @pl.when(s != 0)
    def _():
        for a in range(APT):
            compact_anchor(a)

    plsc.subcore_barrier()
    pltpu.sync_copy(offs_sh, offs_v)

    @pl.when(s == 0)
    def _():
        for a in range(APT):
            compact_anchor(a)

    def do_anchor(a, acc):
        i = wid * APT + a
        np_i = sread(npos_v, i)
        nn_i = sread(nneg_v, i)
        valid = jnp.logical_and(np_i > 0, nn_i > 0)

        def run(acc):
            m = jnp.maximum(np_i, nn_i)
            kmin = jnp.maximum(jnp.minimum(np_i, nn_i), 1)
            unequal = np_i != nn_i
            need = jnp.logical_and(unequal, kmin > 1)
            pos_rand = jnp.logical_and(unequal, np_i < nn_i)
            neg_rand = jnp.logical_and(unequal, nn_i < np_i)
            av = jnp.full((L,), a, jnp.int32)

            def stream_terms(acc):
                o = sread(offs_v, i)
                mb = _mask_of(kmin)
                kmv = kmin - 1
                row0 = pl.multiple_of((o // 1024) * 8, 8)
                base = row0 * 128
                pltpu.sync_copy(w_hbm.at[pl.ds(row0, WIN128)], win_v)

                def cond(st):
                    k, cnt, _ = st
                    return jnp.logical_and(cnt < m, k < WIN128 * 8)

                def step(st):
                    k, cnt, acc = st
                    w = win_v[k // 8, pl.ds((k & 7) * L, L)]
                    flat = base + k * L + lanes
                    aa = jnp.logical_and((w & mb) <= kmv, flat >= o)
                    ai = aa.astype(jnp.int32)
                    jl = cnt + plsc.cumsum(ai) - 1
                    act = jnp.logical_and(aa, jl < m)
                    v = jnp.minimum(w & mb, N - 1)
                    jc = jnp.clip(jl, 0, N - 1)
                    pidx = jnp.where(pos_rand, v, jc)
                    qidx = jnp.where(pos_rand, jc, v)
                    pv = plsc.load_gather(dpos_b, [av, pidx], mask=act)
                    qv = plsc.load_gather(dneg_b, [av, qidx], mask=act)
                    t = jnp.where(act,
                                  jnp.maximum(pv - qv + MARGIN, 0.0), 0.0)
                    return k + 1, cnt + jnp.sum(ai), acc + t

                _, _, acc = lax.while_loop(cond, step,
                                           (jnp.int32(0), jnp.int32(0), acc))
                return acc

            def det_terms(acc):
                def step(ch, acc):
                    jl = ch * L + lanes
                    act = jl < m
                    jc = jnp.clip(jl, 0, N - 1)
                    zero = jnp.zeros((L,), jnp.int32)
                    pidx = jnp.where(pos_rand, zero, jc)
                    qidx = jnp.where(neg_rand, zero, jc)
                    pv = plsc.load_gather(dpos_b, [av, pidx], mask=act)
                    qv = plsc.load_gather(dneg_b, [av, qidx], mask=act)
                    t = jnp.where(act,
                                  jnp.maximum(pv - qv + MARGIN, 0.0), 0.0)
                    return acc + t

                nch = (m + L - 1) // L
                return lax.fori_loop(0, nch, step, acc)

            return lax.cond(need, stream_terms, det_terms, acc)

        return lax.cond(valid, run, lambda acc: acc, acc)

    acc = lax.fori_loop(0, APT, do_anchor, jnp.zeros((L,), jnp.float32))
    acc_v[...] = acc
    pltpu.sync_copy(acc_v, out_hbm.at[wid])


def _sc_stage(d_mat, targets, npos, nneg, w_stream):
    mesh = plsc.VectorSubcoreMesh(core_axis_name="c", subcore_axis_name="s")
    f = pl.kernel(
        _sc_body,
        out_type=jax.ShapeDtypeStruct((32, L), jnp.float32),
        mesh=mesh,
        compiler_params=pltpu.CompilerParams(needs_layout_passes=False),
        scratch_types=[
            pltpu.VMEM((N,), jnp.int32),
            pltpu.VMEM((N,), jnp.int32),
            pltpu.VMEM((N,), jnp.int32),
            pltpu.VMEM((N,), jnp.int32),
            pltpu.VMEM((N,), jnp.float32),
            pltpu.VMEM((APT, N), jnp.float32),
            pltpu.VMEM((APT, N), jnp.float32),
            pltpu.VMEM((WIN128, 128), jnp.int32),
            pltpu.VMEM((RING128, 128), jnp.int32),
            pltpu.VMEM((L,), jnp.float32),
            pltpu.VMEM_SHARED((N,), jnp.int32),
        ],
    )
    return f(d_mat, targets, npos, nneg, w_stream)


def kernel(samples, targets):
    trow = targets.reshape(N, 1)
    tcol = targets.reshape(1, N)
    d_mat, nposc, nnegc, cnt = _tc_stage(samples, trow, tcol)
    w_stream = jnp.asarray(_W_HOST)
    partial = _sc_stage(d_mat, targets, nposc.reshape(N), nnegc.reshape(N),
                        w_stream)
    return jnp.sum(partial) / cnt[0, 0].astype(jnp.float32)


# speedup vs baseline: 1008.2468x; 1.1264x over previous
import numpy as np
import jax
import jax.numpy as jnp
from jax import lax
from jax.experimental import pallas as pl
from jax.experimental.pallas import tpu as pltpu
from jax.experimental.pallas import tpu_sc as plsc

MARGIN = 0.15
EPS = 1e-6
N = 512
FEAT = 128
L = 16
NCHUNK = N // L
WIN128 = 24
BLK128 = 128
RING128 = 256
W128 = 8320
STREAM_WORDS = W128 * 128
APT = N // 32

_W_HOST = (
    np.random.RandomState(0)
    .randint(0, 2 ** 32, size=STREAM_WORDS, dtype=np.uint32)
    .astype(np.int32)
    .reshape(W128, 128)
)


def _tc_body(s_ref, trow_ref, tcol_ref, d_ref, npos_ref, nneg_ref, cnt_ref):
    s = s_ref[...]
    dn = (((1,), (1,)), ((), ()))
    g = lax.dot_general(s, s, dn, precision=lax.Precision.HIGHEST,
                        preferred_element_type=jnp.float32)
    ssq = s * s
    ones_row = jnp.ones((1, FEAT), jnp.float32)
    sq_row = jnp.sum(ssq, axis=1, keepdims=True)
    sq_col = lax.dot_general(ones_row, ssq, dn,
                             precision=lax.Precision.HIGHEST,
                             preferred_element_type=jnp.float32)
    rs_row = jnp.sum(s, axis=1, keepdims=True)
    rs_col = lax.dot_general(ones_row, s, dn,
                             precision=lax.Precision.HIGHEST,
                             preferred_element_type=jnp.float32)
    d2 = (sq_row + sq_col - 2.0 * g
          + (2.0 * EPS) * (rs_row - rs_col)
          + jnp.float32(FEAT * EPS * EPS))
    d_ref[...] = jnp.sqrt(jnp.maximum(d2, 0.0))

    trow = trow_ref[...]
    tcol = tcol_ref[...]
    rowi = lax.broadcasted_iota(jnp.int32, (N, N), 0)
    colj = lax.broadcasted_iota(jnp.int32, (N, N), 1)
    eq = trow == tcol
    posm = jnp.logical_and(eq, colj > rowi).astype(jnp.float32)
    negm = jnp.logical_not(eq).astype(jnp.float32)
    nposi = jnp.sum(posm, axis=1, keepdims=True).astype(jnp.int32)
    nnegi = jnp.sum(negm, axis=1, keepdims=True).astype(jnp.int32)
    npos_ref[...] = nposi
    nneg_ref[...] = nnegi
    valid = jnp.logical_and(nposi > 0, nnegi > 0)
    m = jnp.where(valid, jnp.maximum(nposi, nnegi), 0)
    cnt_ref[...] = jnp.sum(m, axis=(0, 1), keepdims=True)


def _tc_stage(samples, trow, tcol):
    return pl.pallas_call(
        _tc_body,
        out_shape=[
            jax.ShapeDtypeStruct((N, N), jnp.float32),
            jax.ShapeDtypeStruct((N, 1), jnp.int32),
            jax.ShapeDtypeStruct((N, 1), jnp.int32),
            jax.ShapeDtypeStruct((1, 1), jnp.int32),
        ],
    )(samples, trow, tcol)


def _mask_of(kmin):
    v = kmin - 1
    v = v | (v >> 1)
    v = v | (v >> 2)
    v = v | (v >> 4)
    v = v | (v >> 8)
    return v | (v >> 16)


def _sc_body(d_hbm, tgt_hbm, npos_hbm, nneg_hbm, w_hbm, out_hbm,
             tgt_v, npos_v, nneg_v, offs_v, drow_v, dpos_b, dneg_b,
             win_v, ring_v, acc_v, offs_sh):
    c = lax.axis_index("c")
    s = lax.axis_index("s")
    wid = c * 16 + s
    lanes = lax.iota(jnp.int32, L)

    def sread(ref, i):
        base = (i // L) * L
        v = ref[pl.ds(base, L)]
        return jnp.sum(jnp.where(lanes == i - base, v, jnp.zeros_like(v)))

    pltpu.sync_copy(tgt_hbm, tgt_v)
    pltpu.sync_copy(npos_hbm, npos_v)
    pltpu.sync_copy(nneg_hbm, nneg_v)

    def compact_anchor(a):
        i = wid * APT + a
        np_i = sread(npos_v, i)
        nn_i = sread(nneg_v, i)
        lbl = sread(tgt_v, i)
        valid = jnp.logical_and(np_i > 0, nn_i > 0)

        @pl.when(valid)
        def _():
            pltpu.sync_copy(d_hbm.at[i], drow_v)
            av = jnp.full((L,), a, jnp.int32)

            def comp(ch, carry):
                pc, nc = carry
                t = tgt_v[pl.ds(ch * L, L)]
                col = ch * L + lanes
                pm = jnp.logical_and(t == lbl, col > i)
                nm = t != lbl
                dv = drow_v[pl.ds(ch * L, L)]
                pmi = pm.astype(jnp.int32)
                nmi = nm.astype(jnp.int32)
                pr = plsc.cumsum(pmi)
                nr = plsc.cumsum(nmi)
                plsc.store_scatter(dpos_b, [av, pc + pr - 1], dv, mask=pm)
                plsc.store_scatter(dneg_b, [av, nc + nr - 1], dv, mask=nm)
                return pc + jnp.sum(pmi), nc + jnp.sum(nmi)

            lax.fori_loop(0, NCHUNK, comp, (jnp.int32(0), jnp.int32(0)))

    def ring_chunk(q):
        return ring_v[(q // 8) & (RING128 - 1), pl.ds((q & 7) * L, L)]

    @pl.when(s == 0)
    def _scan():
        for b in range(2):
            pltpu.sync_copy(w_hbm.at[pl.ds(b * BLK128, BLK128)],
                            ring_v.at[pl.ds(b * BLK128, BLK128)])

        def anchor_step(i, carry):
            o, loaded = carry
            np_i = sread(npos_v, i)
            nn_i = sread(nneg_v, i)
            valid = jnp.logical_and(np_i > 0, nn_i > 0)
            kmin = jnp.maximum(jnp.minimum(np_i, nn_i), 1)
            need = jnp.logical_and(valid,
                                   jnp.logical_and(np_i != nn_i, kmin > 1))
            plsc.store_scatter(offs_v, [jnp.full((L,), i, jnp.int32)],
                               jnp.full((L,), o, jnp.int32), mask=lanes == 0)

            def consume(carry):
                o, loaded = carry
                d = jnp.maximum(np_i, nn_i)
                mb = _mask_of(kmin)
                kmv = kmin - 1
                q0 = o // L
                phi = o - q0 * L
                r128 = o // 128

                def refill_cond(ld):
                    return r128 + WIN128 > ld

                def refill(ld):
                    slot = (ld // BLK128) % (RING128 // BLK128)
                    pltpu.sync_copy(
                        w_hbm.at[pl.ds(pl.multiple_of(ld, BLK128), BLK128)],
                        ring_v.at[pl.ds(pl.multiple_of(slot * BLK128,
                                                       BLK128), BLK128)])
                    return ld + BLK128

                loaded = lax.while_loop(refill_cond, refill, loaded)

                w = ring_chunk(q0)
                a0 = jnp.logical_and((w & mb) <= kmv, lanes < phi)
                dd_v = jnp.full((L,), d, jnp.int32) \
                    + plsc.all_reduce_population_count(a0)

                def cond(st):
                    k, _, cnt = st
                    return jnp.logical_and(k < q0 + 80,
                                           jnp.any(cnt < dd_v))

                def step(st):
                    k, _, cnt = st
                    ca = jnp.zeros((L,), jnp.int32)
                    for u in range(8):
                        w = ring_chunk(k + u)
                        au = (w & mb) <= kmv
                        ca = ca + plsc.all_reduce_population_count(au)
                    return k + 8, cnt, cnt + ca

                zc = jnp.zeros((L,), jnp.int32)
                k, cprev, _ = lax.while_loop(cond, step, (q0, zc, zc))

                fp = jnp.full((L,), o, jnp.int32)
                bc = cprev
                for u in range(8):
                    w = ring_chunk(k - 8 + u)
                    au = (w & mb) <= kmv
                    ai = au.astype(jnp.int32)
                    ca = plsc.all_reduce_population_count(au)
                    cs = plsc.cumsum(ai)
                    hit = jnp.logical_and(au, cs == dd_v - bc)
                    lane = plsc.all_reduce_ffs(hit)
                    crossed = jnp.logical_and(bc < dd_v, bc + ca >= dd_v)
                    fp = jnp.where(crossed, (k - 8 + u) * L + lane + 1, fp)
                    bc = bc + ca
                return jnp.max(fp), loaded

            return lax.cond(need, consume, lambda cr: cr, (o, loaded))

        lax.fori_loop(0, N, anchor_step,
                      (jnp.int32(0), jnp.int32(2 * BLK128)))
        pltpu.sync_copy(offs_v, offs_sh)

    @pl.when(s != 0)
    def _():
        for a in range(APT):
            compact_anchor(a)

    plsc.subcore_barrier()
    pltpu.sync_copy(offs_sh, offs_v)

    @pl.when(s == 0)
    def _():
        for a in range(APT):
            compact_anchor(a)

    def do_anchor(a, acc):
        i = wid * APT + a
        np_i = sread(npos_v, i)
        nn_i = sread(nneg_v, i)
        valid = jnp.logical_and(np_i > 0, nn_i > 0)

        def run(acc):
            m = jnp.maximum(np_i, nn_i)
            kmin = jnp.maximum(jnp.minimum(np_i, nn_i), 1)
            unequal = np_i != nn_i
            need = jnp.logical_and(unequal, kmin > 1)
            pos_rand = jnp.logical_and(unequal, np_i < nn_i)
            neg_rand = jnp.logical_and(unequal, nn_i < np_i)
            av = jnp.full((L,), a, jnp.int32)

            def stream_terms(acc):
                o = sread(offs_v, i)
                mb = _mask_of(kmin)
                kmv = kmin - 1
                row0 = pl.multiple_of((o // 1024) * 8, 8)
                base = row0 * 128
                pltpu.sync_copy(w_hbm.at[pl.ds(row0, WIN128)], win_v)

                def cond(st):
                    k, cnt, _ = st
                    return jnp.logical_and(cnt < m, k < WIN128 * 8)

                def step(st):
                    k, cnt, acc = st
                    w = win_v[k // 8, pl.ds((k & 7) * L, L)]
                    flat = base + k * L + lanes
                    aa = jnp.logical_and((w & mb) <= kmv, flat >= o)
                    ai = aa.astype(jnp.int32)
                    jl = cnt + plsc.cumsum(ai) - 1
                    act = jnp.logical_and(aa, jl < m)
                    v = jnp.minimum(w & mb, N - 1)
                    jc = jnp.clip(jl, 0, N - 1)
                    pidx = jnp.where(pos_rand, v, jc)
                    qidx = jnp.where(pos_rand, jc, v)
                    pv = plsc.load_gather(dpos_b, [av, pidx], mask=act)
                    qv = plsc.load_gather(dneg_b, [av, qidx], mask=act)
                    t = jnp.where(act,
                                  jnp.maximum(pv - qv + MARGIN, 0.0), 0.0)
                    return k + 1, cnt + jnp.sum(ai), acc + t

                _, _, acc = lax.while_loop(cond, step,
                                           (jnp.int32(0), jnp.int32(0), acc))
                return acc

            def det_terms(acc):
                def step(ch, acc):
                    jl = ch * L + lanes
                    act = jl < m
                    jc = jnp.clip(jl, 0, N - 1)
                    zero = jnp.zeros((L,), jnp.int32)
                    pidx = jnp.where(pos_rand, zero, jc)
                    qidx = jnp.where(neg_rand, zero, jc)
                    pv = plsc.load_gather(dpos_b, [av, pidx], mask=act)
                    qv = plsc.load_gather(dneg_b, [av, qidx], mask=act)
                    t = jnp.where(act,
                                  jnp.maximum(pv - qv + MARGIN, 0.0), 0.0)
                    return acc + t

                nch = (m + L - 1) // L
                return lax.fori_loop(0, nch, step, acc)

            return lax.cond(need, stream_terms, det_terms, acc)

        return lax.cond(valid, run, lambda acc: acc, acc)

    acc = lax.fori_loop(0, APT, do_anchor, jnp.zeros((L,), jnp.float32))
    acc_v[...] = acc
    pltpu.sync_copy(acc_v, out_hbm.at[wid])


def _sc_stage(d_mat, targets, npos, nneg, w_stream):
    mesh = plsc.VectorSubcoreMesh(core_axis_name="c", subcore_axis_name="s")
    f = pl.kernel(
        _sc_body,
        out_type=jax.ShapeDtypeStruct((32, L), jnp.float32),
        mesh=mesh,
        compiler_params=pltpu.CompilerParams(needs_layout_passes=False),
        scratch_types=[
            pltpu.VMEM((N,), jnp.int32),
            pltpu.VMEM((N,), jnp.int32),
            pltpu.VMEM((N,), jnp.int32),
            pltpu.VMEM((N,), jnp.int32),
            pltpu.VMEM((N,), jnp.float32),
            pltpu.VMEM((APT, N), jnp.float32),
            pltpu.VMEM((APT, N), jnp.float32),
            pltpu.VMEM((WIN128, 128), jnp.int32),
            pltpu.VMEM((RING128, 128), jnp.int32),
            pltpu.VMEM((L,), jnp.float32),
            pltpu.VMEM_SHARED((N,), jnp.int32),
        ],
    )
    return f(d_mat, targets, npos, nneg, w_stream)


def kernel(samples, targets):
    trow = targets.reshape(N, 1)
    tcol = targets.reshape(1, N)
    d_mat, nposc, nnegc, cnt = _tc_stage(samples, trow, tcol)
    w_stream = jnp.asarray(_W_HOST)
    partial = _sc_stage(d_mat, targets, nposc.reshape(N), nnegc.reshape(N),
                        w_stream)
    return jnp.sum(partial) / cnt[0, 0].astype(jnp.float32)
